# baseline (speedup 1.0000x reference)
import functools
import jax
import jax.numpy as jnp
from jax import lax
from jax.experimental import pallas as pl
from jax.experimental.pallas import tpu as pltpu
from jax.experimental.pallas import tpu_sc as plsc

IN_F = 1024
OUT_F = 1024
RANK = 16
NE = 16
SCALING = 2.0
TB = 512
N_TOK = 2048
NW = 32
TPW = N_TOK // NW


def _logits_kernel(x_ref, rwt_ref, out_ref):
    out_ref[...] = jnp.dot(x_ref[...], rwt_ref[...],
                           preferred_element_type=jnp.float32)


def _lane_bcast(v, lane):
    idx = jnp.full((NE,), lane, jnp.int32)
    return lax.gather(
        v, idx[:, None],
        dimension_numbers=lax.GatherDimensionNumbers(
            offset_dims=(), collapsed_slice_dims=(0,), start_index_map=(0,)),
        slice_sizes=(1,), mode=lax.GatherScatterMode.PROMISE_IN_BOUNDS)


def _sc_router(lg_hbm, w_hbm, lg_v, w_v):
    wid = lax.axis_index("s") * 2 + lax.axis_index("c")
    base = wid * TPW
    pltpu.sync_copy(lg_hbm.at[pl.ds(base, TPW)], lg_v)
    iota = lax.iota(jnp.int32, NE)

    def body(i, carry):
        l = lg_v[i]
        k_sorted, v_sorted = plsc.sort_key_val(l, iota, descending=True)
        top = _lane_bcast(k_sorted, 0)
        e = jnp.exp(k_sorted - top)
        e2 = _lane_bcast(e, 1)
        wsorted = jnp.where(iota == 0, 1.0, jnp.where(iota == 1, e, 0.0))
        wsorted = wsorted / (1.0 + e2)
        row = jnp.full((NE,), i, jnp.int32)
        plsc.store_scatter(w_v, [row, v_sorted], wsorted)
        return carry

    lax.fori_loop(0, TPW, body, 0)
    pltpu.sync_copy(w_v, w_hbm.at[pl.ds(base, TPW)])


def _main_kernel(x_ref, w_ref, wcat_ref, bb_ref, bf_ref, out_ref):
    xb16 = x_ref[...].astype(jnp.bfloat16)
    res = jnp.dot(xb16, wcat_ref[...], preferred_element_type=jnp.float32)
    base = res[:, :OUT_F]
    h = res[:, OUT_F:]
    er = jax.lax.broadcasted_iota(jnp.int32, (NE, NE * RANK), 0)
    ec = jax.lax.broadcasted_iota(jnp.int32, (NE, NE * RANK), 1)
    expand = (ec // RANK == er).astype(jnp.float32)
    hw = (h * jnp.dot(w_ref[...], expand,
                      preferred_element_type=jnp.float32)).astype(jnp.bfloat16)
    lora = jnp.dot(hw, bf_ref[...], preferred_element_type=jnp.float32)
    out_ref[...] = base + bb_ref[...] + lora


def kernel(x, base_W, base_b, router_W, lora_A, lora_B):
    orig_shape = x.shape
    x_flat = x.reshape(-1, IN_F)
    n_tok = x_flat.shape[0]

    rwt = router_W.T
    bwt = base_W.T.astype(jnp.bfloat16)
    at = lora_A.reshape(NE * RANK, IN_F).T.astype(jnp.bfloat16)
    wcat = jnp.concatenate([bwt, at], axis=1)
    bf = (lora_B.transpose(0, 2, 1).reshape(NE * RANK, OUT_F) * SCALING).astype(jnp.bfloat16)
    bb = base_b.reshape(1, OUT_F)

    logits = pl.pallas_call(
        _logits_kernel,
        grid=(n_tok // TB,),
        in_specs=[
            pl.BlockSpec((TB, IN_F), lambda i: (i, 0)),
            pl.BlockSpec((IN_F, NE), lambda i: (0, 0)),
        ],
        out_specs=pl.BlockSpec((TB, NE), lambda i: (i, 0)),
        out_shape=jax.ShapeDtypeStruct((n_tok, NE), jnp.float32),
        compiler_params=pltpu.CompilerParams(
            dimension_semantics=("arbitrary",),
        ),
    )(x_flat, rwt)

    mesh = plsc.VectorSubcoreMesh(core_axis_name="c", subcore_axis_name="s")
    w = pl.kernel(
        _sc_router,
        out_type=jax.ShapeDtypeStruct((n_tok, NE), jnp.float32),
        mesh=mesh,
        scratch_types=[
            pltpu.VMEM((TPW, NE), jnp.float32),
            pltpu.VMEM((TPW, NE), jnp.float32),
        ],
        compiler_params=pltpu.CompilerParams(needs_layout_passes=False),
    )(logits)

    out = pl.pallas_call(
        _main_kernel,
        grid=(n_tok // TB,),
        in_specs=[
            pl.BlockSpec((TB, IN_F), lambda i: (i, 0)),
            pl.BlockSpec((TB, NE), lambda i: (i, 0)),
            pl.BlockSpec((IN_F, OUT_F + NE * RANK), lambda i: (0, 0)),
            pl.BlockSpec((1, OUT_F), lambda i: (0, 0)),
            pl.BlockSpec((NE * RANK, OUT_F), lambda i: (0, 0)),
        ],
        out_specs=pl.BlockSpec((TB, OUT_F), lambda i: (i, 0)),
        out_shape=jax.ShapeDtypeStruct((n_tok, OUT_F), x.dtype),
        compiler_params=pltpu.CompilerParams(
            dimension_semantics=("arbitrary",),
        ),
    )(x_flat, w, wcat, bb, bf)
    return out.reshape(*orig_shape[:-1], OUT_F)


# speedup vs baseline: 1.6133x; 1.6133x over previous
import jax
import jax.numpy as jnp
from jax.experimental import pallas as pl
from jax.experimental.pallas import tpu as pltpu

IN_F = 1024
OUT_F = 1024
RANK = 16
NE = 16
SCALING = 2.0
TB = 512


def _routing_weights(logits):
    m = jnp.max(logits, axis=-1, keepdims=True)
    e = jnp.exp(logits - m)
    iota = jax.lax.broadcasted_iota(jnp.int32, e.shape, 1)
    i1 = jnp.min(jnp.where(e == 1.0, iota, NE), axis=-1, keepdims=True)
    oh1 = iota == i1
    em = jnp.where(oh1, -1.0, e)
    m2 = jnp.max(em, axis=-1, keepdims=True)
    i2 = jnp.min(jnp.where(em == m2, iota, NE), axis=-1, keepdims=True)
    sel = oh1 | (iota == i2)
    return jnp.where(sel, e, 0.0) / (1.0 + m2)


def _fused_kernel(x_ref, bwt_ref, bb_ref, rwt_ref, at_ref, bf_ref, out_ref):
    xb = x_ref[...]
    logits = jnp.dot(xb, rwt_ref[...], preferred_element_type=jnp.float32)
    w = _routing_weights(logits)
    xb16 = xb.astype(jnp.bfloat16)
    base = jnp.dot(xb16, bwt_ref[...], preferred_element_type=jnp.float32)
    h = jnp.dot(xb16, at_ref[...], preferred_element_type=jnp.float32)
    er = jax.lax.broadcasted_iota(jnp.int32, (NE, NE * RANK), 0)
    ec = jax.lax.broadcasted_iota(jnp.int32, (NE, NE * RANK), 1)
    expand = (ec // RANK == er).astype(jnp.float32)
    hw = (h * jnp.dot(w, expand,
                      preferred_element_type=jnp.float32)).astype(jnp.bfloat16)
    lora = jnp.dot(hw, bf_ref[...], preferred_element_type=jnp.float32)
    out_ref[...] = base + bb_ref[...] + lora


def kernel(x, base_W, base_b, router_W, lora_A, lora_B):
    orig_shape = x.shape
    x_flat = x.reshape(-1, IN_F)
    n_tok = x_flat.shape[0]
    grid = (n_tok // TB,)

    bwt = base_W.T.astype(jnp.bfloat16)
    at = lora_A.reshape(NE * RANK, IN_F).T.astype(jnp.bfloat16)
    rwt = router_W.T
    bf = (lora_B.transpose(0, 2, 1).reshape(NE * RANK, OUT_F) * SCALING).astype(jnp.bfloat16)
    bb = base_b.reshape(1, OUT_F)

    out = pl.pallas_call(
        _fused_kernel,
        grid=grid,
        in_specs=[
            pl.BlockSpec((TB, IN_F), lambda i: (i, 0)),
            pl.BlockSpec((IN_F, OUT_F), lambda i: (0, 0)),
            pl.BlockSpec((1, OUT_F), lambda i: (0, 0)),
            pl.BlockSpec((IN_F, NE), lambda i: (0, 0)),
            pl.BlockSpec((IN_F, NE * RANK), lambda i: (0, 0)),
            pl.BlockSpec((NE * RANK, OUT_F), lambda i: (0, 0)),
        ],
        out_specs=pl.BlockSpec((TB, OUT_F), lambda i: (i, 0)),
        out_shape=jax.ShapeDtypeStruct((n_tok, OUT_F), x.dtype),
        compiler_params=pltpu.CompilerParams(
            dimension_semantics=("arbitrary",),
        ),
    )(x_flat, bwt, bb, rwt, at, bf)
    return out.reshape(*orig_shape[:-1], OUT_F)


# speedup vs baseline: 1.8368x; 1.1385x over previous
import jax
import jax.numpy as jnp
from jax.experimental import pallas as pl
from jax.experimental.pallas import tpu as pltpu

IN_F = 1024
OUT_F = 1024
RANK = 16
NE = 16
SCALING = 2.0
TB = 512

_DN_T = (((1,), (1,)), ((), ()))


def _routing_weights(logits):
    m = jnp.max(logits, axis=-1, keepdims=True)
    e = jnp.exp(logits - m)
    iota = jax.lax.broadcasted_iota(jnp.int32, e.shape, 1)
    i1 = jnp.min(jnp.where(e == 1.0, iota, NE), axis=-1, keepdims=True)
    oh1 = iota == i1
    em = jnp.where(oh1, -1.0, e)
    m2 = jnp.max(em, axis=-1, keepdims=True)
    i2 = jnp.min(jnp.where(em == m2, iota, NE), axis=-1, keepdims=True)
    sel = oh1 | (iota == i2)
    return jnp.where(sel, e, 0.0) / (1.0 + m2)


def _fused_kernel(x_ref, bw_ref, bb_ref, rw_ref, a_ref, bf_ref, out_ref):
    xb = x_ref[...]
    logits = jax.lax.dot_general(xb, rw_ref[...], _DN_T,
                                 preferred_element_type=jnp.float32)
    w = _routing_weights(logits)
    xb16 = xb.astype(jnp.bfloat16)
    base = jax.lax.dot_general(xb16, bw_ref[...], _DN_T,
                               preferred_element_type=jnp.float32)
    h = jax.lax.dot_general(xb16, a_ref[...], _DN_T,
                            preferred_element_type=jnp.float32)
    er = jax.lax.broadcasted_iota(jnp.int32, (NE, NE * RANK), 0)
    ec = jax.lax.broadcasted_iota(jnp.int32, (NE, NE * RANK), 1)
    expand = (ec // RANK == er).astype(jnp.float32)
    hw = (h * jnp.dot(w, expand,
                      preferred_element_type=jnp.float32)).astype(jnp.bfloat16)
    lora = jnp.dot(hw, bf_ref[...], preferred_element_type=jnp.float32)
    out_ref[...] = base + bb_ref[...] + lora


def kernel(x, base_W, base_b, router_W, lora_A, lora_B):
    orig_shape = x.shape
    x_flat = x.reshape(-1, IN_F)
    n_tok = x_flat.shape[0]
    grid = (n_tok // TB,)

    bw16 = base_W.astype(jnp.bfloat16)
    a16 = lora_A.reshape(NE * RANK, IN_F).astype(jnp.bfloat16)
    bf = (lora_B.transpose(0, 2, 1).reshape(NE * RANK, OUT_F) * SCALING).astype(jnp.bfloat16)
    bb = base_b.reshape(1, OUT_F)

    out = pl.pallas_call(
        _fused_kernel,
        grid=grid,
        in_specs=[
            pl.BlockSpec((TB, IN_F), lambda i: (i, 0)),
            pl.BlockSpec((OUT_F, IN_F), lambda i: (0, 0)),
            pl.BlockSpec((1, OUT_F), lambda i: (0, 0)),
            pl.BlockSpec((NE, IN_F), lambda i: (0, 0)),
            pl.BlockSpec((NE * RANK, IN_F), lambda i: (0, 0)),
            pl.BlockSpec((NE * RANK, OUT_F), lambda i: (0, 0)),
        ],
        out_specs=pl.BlockSpec((TB, OUT_F), lambda i: (i, 0)),
        out_shape=jax.ShapeDtypeStruct((n_tok, OUT_F), x.dtype),
        compiler_params=pltpu.CompilerParams(
            dimension_semantics=("arbitrary",),
        ),
    )(x_flat, bw16, bb, router_W, a16, bf)
    return out.reshape(*orig_shape[:-1], OUT_F)
